# baseline (speedup 1.0000x reference)
import functools

import jax
import jax.numpy as jnp
from jax import lax
from jax.experimental import pallas as pl
from jax.experimental.pallas import tpu as pltpu
from jax.experimental.pallas import tpu_sc as plsc

EDGE_ATTR_DIM = 512
HIDDEN_DIM = 16
N_EDGES = 3_200_000

NC = 2
NS = 16
NW = NC * NS
LANES = 16

CHUNK = 2560
TPC = CHUNK // 128
NTILES = N_EDGES // 128
NCHUNKS = N_EDGES // CHUNK
NK_BASE = NCHUNKS // NW
NK_REM = NCHUNKS % NW
NBUF = 2


def _make_gather():
    mesh = plsc.VectorSubcoreMesh(
        core_axis_name="c", subcore_axis_name="s", num_cores=NC, num_subcores=NS
    )

    @functools.partial(
        pl.kernel,
        out_type=jax.ShapeDtypeStruct((2, NTILES, 8, 128), jnp.float32),
        mesh=mesh,
        scratch_types=[
            pltpu.VMEM((HIDDEN_DIM, EDGE_ATTR_DIM), jnp.float32),
            pltpu.VMEM((NBUF, CHUNK), jnp.int32),
            pltpu.VMEM((NBUF, TPC, 8, 128), jnp.float32),
            pltpu.VMEM((NBUF, TPC, 8, 128), jnp.float32),
            [pltpu.SemaphoreType.DMA] * NBUF,
            [pltpu.SemaphoreType.DMA] * NBUF,
            [pltpu.SemaphoreType.DMA] * NBUF,
        ],
        compiler_params=pltpu.CompilerParams(
            use_tc_tiling_on_sc=False, needs_layout_passes=False
        ),
    )
    def gather_kernel(tab_t_hbm, idx_hbm, out_hbm, tab_v, idx_v, buf0, buf1,
                      isems, osems0, osems1):
        wid = lax.axis_index("s") * NC + lax.axis_index("c")
        nk = NK_BASE + (wid < NK_REM).astype(jnp.int32)

        def start_idx(step, b):
            base = (wid + step * NW) * CHUNK
            pltpu.async_copy(idx_hbm.at[pl.ds(base, CHUNK)], idx_v.at[b], isems[b])

        def wait_idx(b):
            pltpu.make_async_copy(
                idx_hbm.at[pl.ds(0, CHUNK)], idx_v.at[b], isems[b]
            ).wait()

        def start_store(step, b):
            tb = (wid + step * NW) * TPC
            pltpu.async_copy(
                buf0.at[b], out_hbm.at[0, pl.ds(tb, TPC)], osems0[b]
            )
            pltpu.async_copy(
                buf1.at[b], out_hbm.at[1, pl.ds(tb, TPC)], osems1[b]
            )

        def wait_store(b):
            pltpu.make_async_copy(
                buf0.at[b], out_hbm.at[0, pl.ds(0, TPC)], osems0[b]
            ).wait()
            pltpu.make_async_copy(
                buf1.at[b], out_hbm.at[1, pl.ds(0, TPC)], osems1[b]
            ).wait()

        pltpu.sync_copy(tab_t_hbm, tab_v)

        def compute(b):
            def tile_blk(t, carry):
                d0 = buf0.at[b].at[t]
                d1 = buf1.at[b].at[t]

                ivs = [
                    idx_v[b, pl.ds(t * 128 + j * LANES, LANES)]
                    for j in range(8)
                ]

                def store(j, c, vec):
                    dst = d0 if c < 8 else d1
                    dst[c % 8, pl.ds(j * LANES, LANES)] = vec

                vecs = [
                    plsc.load_gather(tab_v.at[c], [ivs[0]])
                    for c in range(HIDDEN_DIM)
                ]
                for j in range(8):
                    if j < 7:
                        nxt = []
                        for c in range(HIDDEN_DIM):
                            nxt.append(
                                plsc.load_gather(tab_v.at[c], [ivs[j + 1]])
                            )
                            store(j, c, vecs[c])
                        vecs = nxt
                    else:
                        for c in range(HIDDEN_DIM):
                            store(j, c, vecs[c])
                return carry

            lax.fori_loop(0, TPC, tile_blk, 0)

        def do_step(step, b):
            @pl.when(step >= NBUF)
            def _():
                wait_store(b)

            wait_idx(b)
            compute(b)
            start_store(step, b)

            @pl.when(step + NBUF < nk)
            def _():
                start_idx(step + NBUF, b)

        for b in range(NBUF):
            start_idx(b, b)

        def body(i, carry):
            for b in range(NBUF):
                do_step(i * NBUF + b, b)
            return carry

        lax.fori_loop(0, nk // NBUF, body, 0)

        ntail = lax.rem(nk, NBUF)
        for r in range(NBUF - 1):
            @pl.when(ntail > r)
            def _():
                do_step((nk // NBUF) * NBUF + r, r)

        for b in range(NBUF):
            wait_store(b)

    return gather_kernel


@functools.lru_cache(maxsize=1)
def _gather():
    return _make_gather()


def kernel(edge_attr, table):
    out_phys = _gather()(table.T, edge_attr)
    return out_phys.transpose(1, 3, 0, 2).reshape(N_EDGES, HIDDEN_DIM)


# speedup vs baseline: 1.0174x; 1.0174x over previous
import functools

import jax
import jax.numpy as jnp
from jax import lax
from jax.experimental import pallas as pl
from jax.experimental.pallas import tpu as pltpu
from jax.experimental.pallas import tpu_sc as plsc

EDGE_ATTR_DIM = 512
HIDDEN_DIM = 16
N_EDGES = 3_200_000

NC = 2
NS = 16
NW = NC * NS
LANES = 16

CHUNK = 1024
TPC = CHUNK // 128
NTILES = N_EDGES // 128
NCHUNKS = N_EDGES // CHUNK
NK_BASE = NCHUNKS // NW
NK_REM = NCHUNKS % NW
NBUF = 2


def _make_gather():
    mesh = plsc.VectorSubcoreMesh(
        core_axis_name="c", subcore_axis_name="s", num_cores=NC, num_subcores=NS
    )

    @functools.partial(
        pl.kernel,
        out_type=jax.ShapeDtypeStruct((2, NTILES, 8, 128), jnp.float32),
        mesh=mesh,
        scratch_types=[
            pltpu.VMEM((HIDDEN_DIM, EDGE_ATTR_DIM), jnp.float32),
            pltpu.VMEM((NBUF, CHUNK), jnp.int32),
            pltpu.VMEM((NBUF, TPC, 8, 128), jnp.float32),
            pltpu.VMEM((NBUF, TPC, 8, 128), jnp.float32),
            [pltpu.SemaphoreType.DMA] * NBUF,
            [pltpu.SemaphoreType.DMA] * NBUF,
            [pltpu.SemaphoreType.DMA] * NBUF,
        ],
        compiler_params=pltpu.CompilerParams(
            use_tc_tiling_on_sc=False, needs_layout_passes=False
        ),
    )
    def gather_kernel(tab_t_hbm, idx_hbm, out_hbm, tab_v, idx_v, buf0, buf1,
                      isems, osems0, osems1):
        wid = lax.axis_index("s") * NC + lax.axis_index("c")
        nk = NK_BASE + (wid < NK_REM).astype(jnp.int32)

        def start_idx(step, b):
            base = (wid + step * NW) * CHUNK
            pltpu.async_copy(idx_hbm.at[pl.ds(base, CHUNK)], idx_v.at[b], isems[b])

        def wait_idx(b):
            pltpu.make_async_copy(
                idx_hbm.at[pl.ds(0, CHUNK)], idx_v.at[b], isems[b]
            ).wait()

        def start_store(step, b):
            tb = (wid + step * NW) * TPC
            pltpu.async_copy(
                buf0.at[b], out_hbm.at[0, pl.ds(tb, TPC)], osems0[b]
            )
            pltpu.async_copy(
                buf1.at[b], out_hbm.at[1, pl.ds(tb, TPC)], osems1[b]
            )

        def wait_store(b):
            pltpu.make_async_copy(
                buf0.at[b], out_hbm.at[0, pl.ds(0, TPC)], osems0[b]
            ).wait()
            pltpu.make_async_copy(
                buf1.at[b], out_hbm.at[1, pl.ds(0, TPC)], osems1[b]
            ).wait()

        pltpu.sync_copy(tab_t_hbm, tab_v)

        def compute(b):
            def tile_blk(t, carry):
                d0 = buf0.at[b].at[t]
                d1 = buf1.at[b].at[t]

                ivs = [
                    idx_v[b, pl.ds(t * 128 + j * LANES, LANES)]
                    for j in range(8)
                ]

                def store(j, c, vec):
                    dst = d0 if c < 8 else d1
                    dst[c % 8, pl.ds(j * LANES, LANES)] = vec

                vecs = [
                    plsc.load_gather(tab_v.at[c], [ivs[0]])
                    for c in range(HIDDEN_DIM)
                ]
                for j in range(8):
                    if j < 7:
                        nxt = []
                        for c in range(HIDDEN_DIM):
                            nxt.append(
                                plsc.load_gather(tab_v.at[c], [ivs[j + 1]])
                            )
                            store(j, c, vecs[c])
                        vecs = nxt
                    else:
                        for c in range(HIDDEN_DIM):
                            store(j, c, vecs[c])
                return carry

            lax.fori_loop(0, TPC, tile_blk, 0)

        def do_step(step, b):
            @pl.when(step >= NBUF)
            def _():
                wait_store(b)

            wait_idx(b)
            compute(b)
            start_store(step, b)

            @pl.when(step + NBUF < nk)
            def _():
                start_idx(step + NBUF, b)

        for b in range(NBUF):
            start_idx(b, b)

        def body(i, carry):
            for b in range(NBUF):
                do_step(i * NBUF + b, b)
            return carry

        lax.fori_loop(0, nk // NBUF, body, 0)

        ntail = lax.rem(nk, NBUF)
        for r in range(NBUF - 1):
            @pl.when(ntail > r)
            def _():
                do_step((nk // NBUF) * NBUF + r, r)

        for b in range(NBUF):
            wait_store(b)

    return gather_kernel


@functools.lru_cache(maxsize=1)
def _gather():
    return _make_gather()


def kernel(edge_attr, table):
    out_phys = _gather()(table.T, edge_attr)
    return out_phys.transpose(1, 3, 0, 2).reshape(N_EDGES, HIDDEN_DIM)
